# baseline (speedup 1.0000x reference)
import functools

import jax
import jax.numpy as jnp
from jax import lax
from jax.experimental import pallas as pl
from jax.experimental.pallas import tpu as pltpu
from jax.experimental.pallas import tpu_sc as plsc


def _gather_t(hist: int, batch: int, d: int, dpad: int):
    info = plsc.get_sparse_core_info()
    nc, ns, nl = info.num_cores, info.num_subcores, info.num_lanes
    nw = nc * ns
    bw = batch // nw
    assert bw % 128 == 0 and d % nl == 0
    mesh = plsc.VectorSubcoreMesh(core_axis_name="c", subcore_axis_name="s")

    @functools.partial(
        pl.kernel,
        mesh=mesh,
        compiler_params=pltpu.CompilerParams(
            use_tc_tiling_on_sc=True, needs_layout_passes=False),
        out_type=jax.ShapeDtypeStruct((hist, d, batch), jnp.float32),
        scratch_types=[
            pltpu.VMEM((hist, bw), jnp.int32),
            pltpu.VMEM((2, bw, dpad), jnp.float32),
            pltpu.VMEM((2, d, bw), jnp.float32),
            pltpu.SemaphoreType.DMA((2,)),
            pltpu.SemaphoreType.DMA((2,)),
            pltpu.SemaphoreType.DMA,
        ],
    )
    def k(idx_hbm, table_hbm, out_hbm, idx_v, rows_v, tout_v, gsem, wsem, isem):
        wid = lax.axis_index("s") * nc + lax.axis_index("c")
        bcol = pl.multiple_of(wid * bw, 128)
        pltpu.async_copy(
            idx_hbm.at[:, pl.ds(bcol, bw)], idx_v, isem).wait()

        def gather_start(h, p):
            return pltpu.async_copy(
                table_hbm.at[idx_v.at[h]], rows_v.at[p], gsem.at[p])

        def gather_wait(h, p):
            pltpu.make_async_copy(
                table_hbm.at[idx_v.at[h]], rows_v.at[p], gsem.at[p]).wait()

        def write_start(h, p):
            return pltpu.async_copy(
                tout_v.at[p], out_hbm.at[h, :, pl.ds(bcol, bw)], wsem.at[p])

        def write_wait(h, p):
            pltpu.make_async_copy(
                tout_v.at[p], out_hbm.at[h, :, pl.ds(bcol, bw)],
                wsem.at[p]).wait()

        iot = lax.iota(jnp.int32, nl)
        perms = [(iot + s) & (nl - 1) for s in range(nl)]
        dcols = [iot + dg * nl for dg in range(d // nl)]

        def transpose(p):
            src = rows_v.at[p]
            dst = tout_v.at[p]

            def blk(bg, _):
                for dg in range(d // nl):
                    for s in range(nl):
                        bvec = perms[s] + bg * nl
                        vec = plsc.load_gather(src, [bvec, dcols[dg]])
                        plsc.store_scatter(dst, [dcols[dg], bvec], vec)
                return 0

            lax.fori_loop(0, bw // nl, blk, 0)

        gather_start(0, 0)
        gather_start(1, 1)
        gather_wait(0, 0)
        transpose(0)
        write_start(0, 0)

        def pair(j2, _):
            for b2 in range(2):
                j = j2 * 2 + b2
                p, q = b2, 1 - b2
                write_wait(j - 2, p)
                gather_start(j, p)
                gather_wait(j - 1, q)
                transpose(q)
                write_start(j - 1, q)
            return 0

        lax.fori_loop(1, hist // 2, pair, 0)

        last = hist - 1
        gather_wait(last, last % 2)
        transpose(last % 2)
        write_start(last, last % 2)
        write_wait(last - 1, (last - 1) % 2)
        write_wait(last, last % 2)

    return k


def _pad_rows_tc(v, d, dpad, bs):
    def body(t_ref, o_ref):
        o_ref[:, :d] = t_ref[...]

    return pl.pallas_call(
        body,
        grid=(v // bs,),
        in_specs=[pl.BlockSpec((bs, d), lambda i: (i, 0))],
        out_specs=pl.BlockSpec((bs, dpad), lambda i: (i, 0)),
        out_shape=jax.ShapeDtypeStruct((v, dpad), jnp.float32),
    )


def kernel(x, table):
    batch, hist = x.shape
    v, d = table.shape
    dpad = 128
    xt = x.T
    tp = _pad_rows_tc(v, d, dpad, 8000)(table)
    out3 = _gather_t(hist, batch, d, dpad)(xt, tp)
    return out3.transpose(2, 0, 1)


# speedup vs baseline: 1.1252x; 1.1252x over previous
import functools

import jax
import jax.numpy as jnp
from jax import lax
from jax.experimental import pallas as pl
from jax.experimental.pallas import tpu as pltpu
from jax.experimental.pallas import tpu_sc as plsc


def _gather_t(hist: int, batch: int, d: int, dpad: int):
    info = plsc.get_sparse_core_info()
    nc, ns, nl = info.num_cores, info.num_subcores, info.num_lanes
    nw = nc * ns
    bw = batch // nw
    assert bw % 128 == 0 and d % nl == 0
    mesh = plsc.VectorSubcoreMesh(core_axis_name="c", subcore_axis_name="s")

    @functools.partial(
        pl.kernel,
        mesh=mesh,
        compiler_params=pltpu.CompilerParams(
            use_tc_tiling_on_sc=True, needs_layout_passes=False),
        out_type=jax.ShapeDtypeStruct((hist, d, batch), jnp.float32),
        scratch_types=[
            pltpu.VMEM((hist, bw), jnp.int32),
            pltpu.VMEM((2, bw, dpad), jnp.float32),
            pltpu.VMEM((2, d, bw), jnp.float32),
            pltpu.SemaphoreType.DMA((2,)),
            pltpu.SemaphoreType.DMA((2,)),
            pltpu.SemaphoreType.DMA,
        ],
    )
    def k(idx_hbm, table_hbm, out_hbm, idx_v, rows_v, tout_v, gsem, wsem, isem):
        wid = lax.axis_index("s") * nc + lax.axis_index("c")
        bcol = pl.multiple_of(wid * bw, 128)
        pltpu.async_copy(
            idx_hbm.at[:, pl.ds(bcol, bw)], idx_v, isem).wait()

        def gather_start(h, p):
            return pltpu.async_copy(
                table_hbm.at[idx_v.at[h]], rows_v.at[p], gsem.at[p])

        def gather_wait(h, p):
            pltpu.make_async_copy(
                table_hbm.at[idx_v.at[h]], rows_v.at[p], gsem.at[p]).wait()

        def write_start(h, p):
            return pltpu.async_copy(
                tout_v.at[p], out_hbm.at[h, :, pl.ds(bcol, bw)], wsem.at[p])

        def write_wait(h, p):
            pltpu.make_async_copy(
                tout_v.at[p], out_hbm.at[h, :, pl.ds(bcol, bw)],
                wsem.at[p]).wait()

        iot = lax.iota(jnp.int32, nl)
        perms = [(iot + s) & (nl - 1) for s in range(nl)]
        dcols = [iot + dg * nl for dg in range(d // nl)]

        def transpose(p):
            src = rows_v.at[p]
            dst = tout_v.at[p]

            def blk(bg2, _):
                for u in range(2):
                    bg = bg2 * 2 + u
                    for dg in range(d // nl):
                        for s in range(nl):
                            bvec = perms[s] + bg * nl
                            vec = plsc.load_gather(src, [bvec, dcols[dg]])
                            plsc.store_scatter(dst, [dcols[dg], bvec], vec)
                return 0

            lax.fori_loop(0, bw // (2 * nl), blk, 0)

        gather_start(0, 0)
        gather_start(1, 1)
        gather_wait(0, 0)
        transpose(0)
        write_start(0, 0)

        def pair(j2, _):
            for b2 in range(2):
                j = j2 * 2 + b2
                p, q = b2, 1 - b2
                write_wait(j - 2, p)
                gather_start(j, p)
                gather_wait(j - 1, q)
                transpose(q)
                write_start(j - 1, q)
            return 0

        lax.fori_loop(1, hist // 2, pair, 0)

        last = hist - 1
        gather_wait(last, last % 2)
        transpose(last % 2)
        write_start(last, last % 2)
        write_wait(last - 1, (last - 1) % 2)
        write_wait(last, last % 2)

    return k


def _pad_rows_tc(v, d, dpad, bs):
    def body(t_ref, o_ref):
        o_ref[...] = t_ref[...]

    return pl.pallas_call(
        body,
        grid=(v // bs,),
        in_specs=[pl.BlockSpec((bs, d), lambda i: (i, 0))],
        out_specs=pl.BlockSpec((bs, d), lambda i: (i, 0)),
        out_shape=jax.ShapeDtypeStruct((v, dpad), jnp.float32),
    )


def kernel(x, table):
    batch, hist = x.shape
    v, d = table.shape
    dpad = 128
    xt = x.T
    tp = jnp.pad(table, ((0, 0), (0, dpad - d)))
    out3 = _gather_t(hist, batch, d, dpad)(xt, tp)
    return out3.transpose(2, 0, 1)
